# baseline (speedup 1.0000x reference)
import jax
import jax.numpy as jnp
from jax import lax
from jax.experimental import pallas as pl

_LAMB = max(5.0, 1500.0 / 1.001)
_DENOM = 1.0 + _LAMB
_B = 4096
_C = 1000
_BR = 1024
_NBLK = _B // _BR


def _body(cos_ref, tgt_ref, ph_ref, out_ref):
    i = pl.program_id(0)
    cosb = cos_ref[...]
    tgt = tgt_ref[...]
    pt_ = ph_ref[...]
    col = lax.broadcasted_iota(jnp.int32, cosb.shape, 1)
    mask = col == tgt
    m0 = jnp.max(cosb, axis=1, keepdims=True)
    e = jnp.exp(cosb - m0)
    ones = jnp.ones((_C, 1), jnp.float32)
    s0 = lax.dot_general(e, ones, (((1,), (0,)), ((), ())),
                         preferred_element_type=jnp.float32)
    ct = lax.dot_general(jnp.where(mask, cosb, 0.0), ones,
                         (((1,), (0,)), ((), ())),
                         preferred_element_type=jnp.float32)
    mt = ct + (pt_ - ct) / _DENOM
    m = jnp.maximum(m0, mt)
    s = s0 * jnp.exp(m0 - m) - jnp.exp(ct - m) + jnp.exp(mt - m)
    logpt = mt - m - jnp.log(s)
    pt = jnp.exp(logpt)
    omp = 1.0 - pt
    partial = -jnp.sum(omp * omp * logpt, keepdims=True) / _B

    @pl.when(i == 0)
    def _():
        out_ref[...] = jnp.zeros_like(out_ref)

    out_ref[...] += partial


def kernel(cos_theta, phi_theta, xlen, target):
    del xlen
    tgt_col = target.reshape(_B, 1)
    ph_col = jnp.take_along_axis(phi_theta, tgt_col, axis=1)
    r = pl.pallas_call(
        _body,
        grid=(_NBLK,),
        in_specs=[
            pl.BlockSpec((_BR, _C), lambda i: (i, 0)),
            pl.BlockSpec((_BR, 1), lambda i: (i, 0)),
            pl.BlockSpec((_BR, 1), lambda i: (i, 0)),
        ],
        out_specs=pl.BlockSpec((1, 1), lambda i: (0, 0)),
        out_shape=jax.ShapeDtypeStruct((1, 1), jnp.float32),
    )(cos_theta, tgt_col, ph_col)
    return r[0, 0]
